# baseline (speedup 1.0000x reference)
import functools

import jax
import jax.numpy as jnp
from jax import lax
from jax.experimental import pallas as pl
from jax.experimental.pallas import tpu as pltpu
from jax.experimental.pallas import tpu_sc as plsc

NUM_TICKERS = 1000000
DIM = 32
BATCH = 16384
LENGTH = 50

_SC_B = BATCH

_NUM_CORES = 2
_NUM_SUBCORES = 16
_NW = _NUM_CORES * _NUM_SUBCORES
_B_PER_W = _SC_B // _NW
_CK = 8
_N_CK = _B_PER_W // _CK
_K_TRIPLE = _N_CK // 3

_sc_mesh = plsc.VectorSubcoreMesh(core_axis_name="c", subcore_axis_name="s")


@functools.partial(
    pl.kernel,
    out_type=jax.ShapeDtypeStruct((DIM, _SC_B), jnp.float32),
    mesh=_sc_mesh,
    scratch_types=[
        pltpu.VMEM((_N_CK, 16), jnp.int32),
        pltpu.VMEM((3, DIM, _CK * 128), jnp.float32),
        pltpu.VMEM((DIM, _B_PER_W), jnp.float32),
        pltpu.SemaphoreType.DMA,
        pltpu.SemaphoreType.DMA,
        pltpu.SemaphoreType.DMA,
        pltpu.SemaphoreType.DMA,
    ],
    compiler_params=pltpu.CompilerParams(
        use_tc_tiling_on_sc=True, needs_layout_passes=False
    ),
)
def _sc_gather(table_hbm, idx_hbm, emb_hbm, idx_v, slab_v, cols_v,
               sem_i, sem_a, sem_b, sem_c):
    wid = lax.axis_index("s") * _NUM_CORES + lax.axis_index("c")
    base = wid * _B_PER_W
    pltpu.sync_copy(idx_hbm.at[wid], idx_v)

    iota16 = jax.lax.iota(jnp.int32, 16)

    def fire(c, p, sem):
        v16 = idx_v[c]
        for t in range(_CK):
            idx = v16[t]
            al = lax.shift_left(lax.shift_right_logical(idx, 7), 7)
            for r4 in range(4):
                pltpu.make_async_copy(
                    table_hbm.at[pl.ds(r4 * 8, 8),
                                 pl.ds(pl.multiple_of(al, 128), 128)],
                    slab_v.at[p, pl.ds(r4 * 8, 8), pl.ds(t * 128, 128)],
                    sem,
                ).start()

    def drain(p, sem):
        pltpu.make_async_copy(
            table_hbm.at[:, pl.ds(0, _CK * 128)],
            slab_v.at[p],
            sem,
        ).wait()

    def extract(c, p):
        v16 = idx_v[c]
        pv = jnp.full((16,), p, jnp.int32)
        for t in range(_CK):
            idx = v16[t]
            col = jnp.full((16,), t * 128, jnp.int32) + (idx & 127)
            dst = jnp.full((16,), c * _CK + t, jnp.int32)
            for h in range(2):
                rows = iota16 + (h * 16)
                vals = plsc.load_gather(slab_v, [pv, rows, col])
                plsc.store_scatter(cols_v, [rows, dst], vals)

    sems = (sem_a, sem_b, sem_c)
    fire(0, 0, sems[0])
    fire(1, 1, sems[1])

    def triple(k, _):
        c0 = k * 3
        for s in range(3):
            @pl.when(c0 + s + 2 < _N_CK)
            def _(s=s):
                fire(c0 + s + 2, (s + 2) % 3, sems[(s + 2) % 3])

            drain(s, sems[s])
            extract(c0 + s, s)
        return ()

    lax.fori_loop(0, _K_TRIPLE, triple, ())

    for c in range(3 * _K_TRIPLE, _N_CK):
        if c + 2 < _N_CK:
            fire(c + 2, (c + 2) % 3, sems[(c + 2) % 3])
        drain(c % 3, sems[c % 3])
        extract(c, c % 3)

    pltpu.sync_copy(cols_v, emb_hbm.at[:, pl.ds(base, _B_PER_W)])


_LANE_BLOCK = 1024


def _expand_body(delta_ref, emb_ref, out_ref):
    delta = delta_ref[0, 0]
    out_ref[...] = jnp.broadcast_to(
        emb_ref[...][None, :, :] + delta, (LENGTH, DIM, _LANE_BLOCK)
    )


@jax.jit
def _tc_expand(delta, emb_t):
    return pl.pallas_call(
        _expand_body,
        grid=(BATCH // _LANE_BLOCK,),
        in_specs=[
            pl.BlockSpec(memory_space=pltpu.SMEM),
            pl.BlockSpec((DIM, _LANE_BLOCK), lambda i: (0, i)),
        ],
        out_specs=pl.BlockSpec((LENGTH, DIM, _LANE_BLOCK), lambda i: (0, 0, i)),
        out_shape=jax.ShapeDtypeStruct((LENGTH, DIM, BATCH), jnp.float32),
    )(delta, emb_t)


def kernel(ticker_ids, length, table):
    table_t = table.T
    idsr = ticker_ids.astype(jnp.int32).reshape(_NW, _N_CK, _CK)
    ids16 = jnp.concatenate([idsr, idsr], axis=2)
    emb_t = _sc_gather(table_t, ids16)
    delta = (jnp.asarray(length, jnp.float32) - LENGTH).reshape(1, 1)
    out_t = _tc_expand(delta, emb_t)
    return out_t.transpose(2, 0, 1)


# speedup vs baseline: 1.0012x; 1.0012x over previous
import functools

import jax
import jax.numpy as jnp
from jax import lax
from jax.experimental import pallas as pl
from jax.experimental.pallas import tpu as pltpu
from jax.experimental.pallas import tpu_sc as plsc

NUM_TICKERS = 1000000
DIM = 32
BATCH = 16384
LENGTH = 50

_SC_B = BATCH

_NUM_CORES = 2
_NUM_SUBCORES = 16
_NW = _NUM_CORES * _NUM_SUBCORES
_B_PER_W = _SC_B // _NW
_CK = 8
_N_CK = _B_PER_W // _CK
_K_TRIPLE = _N_CK // 3

_sc_mesh = plsc.VectorSubcoreMesh(core_axis_name="c", subcore_axis_name="s")


@functools.partial(
    pl.kernel,
    out_type=jax.ShapeDtypeStruct((DIM, _SC_B), jnp.float32),
    mesh=_sc_mesh,
    scratch_types=[
        pltpu.VMEM((_N_CK, 16), jnp.int32),
        pltpu.VMEM((3, DIM, _CK * 128), jnp.float32),
        pltpu.VMEM((DIM, _B_PER_W), jnp.float32),
        pltpu.SemaphoreType.DMA,
        pltpu.SemaphoreType.DMA,
        pltpu.SemaphoreType.DMA,
        pltpu.SemaphoreType.DMA,
    ],
    compiler_params=pltpu.CompilerParams(
        use_tc_tiling_on_sc=True, needs_layout_passes=False
    ),
)
def _sc_gather(table_hbm, idx_hbm, emb_hbm, idx_v, slab_v, cols_v,
               sem_i, sem_a, sem_b, sem_c):
    wid = lax.axis_index("s") * _NUM_CORES + lax.axis_index("c")
    base = wid * _B_PER_W
    pltpu.sync_copy(idx_hbm.at[wid], idx_v)

    iota16 = jax.lax.iota(jnp.int32, 16)

    def fire(c, p, sem):
        v16 = idx_v[c]
        for t in range(_CK):
            idx = v16[t]
            al = lax.shift_left(lax.shift_right_logical(idx, 7), 7)
            pltpu.make_async_copy(
                table_hbm.at[:, pl.ds(pl.multiple_of(al, 128), 128)],
                slab_v.at[p, :, pl.ds(t * 128, 128)],
                sem,
            ).start()

    def drain(p, sem):
        pltpu.make_async_copy(
            table_hbm.at[:, pl.ds(0, _CK * 128)],
            slab_v.at[p],
            sem,
        ).wait()

    def extract(c, p):
        v16 = idx_v[c]
        pv = jnp.full((16,), p, jnp.int32)
        for t in range(_CK):
            idx = v16[t]
            col = jnp.full((16,), t * 128, jnp.int32) + (idx & 127)
            dst = jnp.full((16,), c * _CK + t, jnp.int32)
            for h in range(2):
                rows = iota16 + (h * 16)
                vals = plsc.load_gather(slab_v, [pv, rows, col])
                plsc.store_scatter(cols_v, [rows, dst], vals)

    sems = (sem_a, sem_b, sem_c)
    fire(0, 0, sems[0])
    fire(1, 1, sems[1])

    def triple(k, _):
        c0 = k * 3
        for s in range(3):
            @pl.when(c0 + s + 2 < _N_CK)
            def _(s=s):
                fire(c0 + s + 2, (s + 2) % 3, sems[(s + 2) % 3])

            drain(s, sems[s])
            extract(c0 + s, s)
        return ()

    lax.fori_loop(0, _K_TRIPLE, triple, ())

    for c in range(3 * _K_TRIPLE, _N_CK):
        if c + 2 < _N_CK:
            fire(c + 2, (c + 2) % 3, sems[(c + 2) % 3])
        drain(c % 3, sems[c % 3])
        extract(c, c % 3)

    pltpu.sync_copy(cols_v, emb_hbm.at[:, pl.ds(base, _B_PER_W)])


_LANE_BLOCK = 1024


def _expand_body(delta_ref, emb_ref, out_ref):
    delta = delta_ref[0, 0]
    out_ref[...] = jnp.broadcast_to(
        emb_ref[...][None, :, :] + delta, (LENGTH, DIM, _LANE_BLOCK)
    )


@jax.jit
def _tc_expand(delta, emb_t):
    return pl.pallas_call(
        _expand_body,
        grid=(BATCH // _LANE_BLOCK,),
        in_specs=[
            pl.BlockSpec(memory_space=pltpu.SMEM),
            pl.BlockSpec((DIM, _LANE_BLOCK), lambda i: (0, i)),
        ],
        out_specs=pl.BlockSpec((LENGTH, DIM, _LANE_BLOCK), lambda i: (0, 0, i)),
        out_shape=jax.ShapeDtypeStruct((LENGTH, DIM, BATCH), jnp.float32),
    )(delta, emb_t)


def kernel(ticker_ids, length, table):
    table_t = table.T
    idsr = ticker_ids.astype(jnp.int32).reshape(_NW, _N_CK, _CK)
    ids16 = jnp.concatenate([idsr, idsr], axis=2)
    emb_t = _sc_gather(table_t, ids16)
    delta = (jnp.asarray(length, jnp.float32) - LENGTH).reshape(1, 1)
    out_t = _tc_expand(delta, emb_t)
    return out_t.transpose(2, 0, 1)


# speedup vs baseline: 1.0082x; 1.0070x over previous
import functools

import jax
import jax.numpy as jnp
from jax import lax
from jax.experimental import pallas as pl
from jax.experimental.pallas import tpu as pltpu
from jax.experimental.pallas import tpu_sc as plsc

NUM_TICKERS = 1000000
DIM = 32
BATCH = 16384
LENGTH = 50

_SC_B = BATCH

_NUM_CORES = 2
_NUM_SUBCORES = 16
_NW = _NUM_CORES * _NUM_SUBCORES
_B_PER_W = _SC_B // _NW
_CK = 8
_N_CK = _B_PER_W // _CK
_K_TRIPLE = _N_CK // 3

_sc_mesh = plsc.VectorSubcoreMesh(core_axis_name="c", subcore_axis_name="s")


@functools.partial(
    pl.kernel,
    out_type=jax.ShapeDtypeStruct((DIM, _SC_B), jnp.float32),
    mesh=_sc_mesh,
    scratch_types=[
        pltpu.VMEM((_N_CK, 16), jnp.int32),
        pltpu.VMEM((3, DIM, _CK * 128), jnp.float32),
        pltpu.VMEM((DIM, _B_PER_W), jnp.float32),
        pltpu.SemaphoreType.DMA,
        pltpu.SemaphoreType.DMA,
        pltpu.SemaphoreType.DMA,
    ],
    compiler_params=pltpu.CompilerParams(
        use_tc_tiling_on_sc=True, needs_layout_passes=False
    ),
)
def _sc_gather(table_hbm, idx_hbm, emb_hbm, idx_v, slab_v, cols_v,
               sem_a, sem_b, sem_c):
    wid = lax.axis_index("s") * _NUM_CORES + lax.axis_index("c")
    base = wid * _B_PER_W
    pltpu.sync_copy(idx_hbm.at[wid], idx_v)

    iota16 = jax.lax.iota(jnp.int32, 16)

    def fire(c, p, sem):
        v16 = idx_v[c]
        for t in range(_CK):
            idx = v16[t]
            al = lax.shift_left(lax.shift_right_logical(idx, 7), 7)
            pltpu.make_async_copy(
                table_hbm.at[:, pl.ds(pl.multiple_of(al, 128), 128)],
                slab_v.at[p, :, pl.ds(t * 128, 128)],
                sem,
            ).start()

    def drain(p, sem):
        pltpu.make_async_copy(
            table_hbm.at[:, pl.ds(0, _CK * 128)],
            slab_v.at[p],
            sem,
        ).wait()

    def extract(c, p):
        v16 = idx_v[c]
        pv = jnp.full((16,), p, jnp.int32)
        for t in range(_CK):
            idx = v16[t]
            col = jnp.full((16,), t * 128, jnp.int32) + (idx & 127)
            dst = jnp.full((16,), c * _CK + t, jnp.int32)
            for h in range(2):
                rows = iota16 + (h * 16)
                vals = plsc.load_gather(slab_v, [pv, rows, col])
                plsc.store_scatter(cols_v, [rows, dst], vals)

    sems = (sem_a, sem_b, sem_c)
    fire(0, 0, sems[0])
    fire(1, 1, sems[1])

    def triple(k, _):
        c0 = k * 3
        for s in range(3):
            @pl.when(c0 + s + 2 < _N_CK)
            def _(s=s):
                fire(c0 + s + 2, (s + 2) % 3, sems[(s + 2) % 3])

            drain(s, sems[s])
            extract(c0 + s, s)
        return ()

    lax.fori_loop(0, _K_TRIPLE, triple, ())

    for c in range(3 * _K_TRIPLE, _N_CK):
        if c + 2 < _N_CK:
            fire(c + 2, (c + 2) % 3, sems[(c + 2) % 3])
        drain(c % 3, sems[c % 3])
        extract(c, c % 3)

    pltpu.sync_copy(cols_v, emb_hbm.at[:, pl.ds(base, _B_PER_W)])


_LANE_BLOCK = 1024


def _expand_body(delta_ref, emb_ref, out_ref):
    delta = delta_ref[0, 0]
    out_ref[...] = jnp.broadcast_to(
        emb_ref[...][None, :, :] + delta, (LENGTH, DIM, _LANE_BLOCK)
    )


@jax.jit
def _tc_expand(delta, emb_t):
    return pl.pallas_call(
        _expand_body,
        grid=(BATCH // _LANE_BLOCK,),
        in_specs=[
            pl.BlockSpec(memory_space=pltpu.SMEM),
            pl.BlockSpec((DIM, _LANE_BLOCK), lambda i: (0, i)),
        ],
        out_specs=pl.BlockSpec((LENGTH, DIM, _LANE_BLOCK), lambda i: (0, 0, i)),
        out_shape=jax.ShapeDtypeStruct((LENGTH, DIM, BATCH), jnp.float32),
    )(delta, emb_t)


def kernel(ticker_ids, length, table):
    table_t = table.T
    idsr = ticker_ids.astype(jnp.int32).reshape(_NW, _N_CK, _CK)
    ids16 = jnp.concatenate([idsr, idsr], axis=2)
    emb_t = _sc_gather(table_t, ids16)
    delta = (jnp.asarray(length, jnp.float32) - LENGTH).reshape(1, 1)
    out_t = _tc_expand(delta, emb_t)
    return out_t.transpose(2, 0, 1)
